# baseline (speedup 1.0000x reference)
import functools

import jax
import jax.numpy as jnp
from jax import lax
from jax.experimental import pallas as pl
from jax.experimental.pallas import tpu as pltpu
from jax.experimental.pallas import tpu_sc as plsc

N = 10000
E = 320000
ELBL = 20000
DIN = 384
D = 128

NC, NS = 2, 16
CH = 128
NBUF = 2
NPAD = 10240
ROWS_PER_TILE = NPAD // NS
N_CHUNKS = 160
E_TILE = N_CHUNKS * CH
E_PAD = E_TILE * NS
GRP = NBUF * CH
LBL_TILE = 640
LBL_PAD = LBL_TILE * NC * NS
LBL_CHUNKS = LBL_TILE // CH

_MESH = plsc.VectorSubcoreMesh(
    core_axis_name="c", subcore_axis_name="s", num_cores=NC, num_subcores=NS)


def _agg_direction(sid, tab, srcf, dstf, out, acc, sidx, didx, rows, gsem,
                   ssem, isem, zrow):
  row0 = sid * ROWS_PER_TILE
  base = sid * E_TILE
  ngroups = N_CHUNKS // NBUF
  pltpu.sync_copy(zrow, acc.at[pl.ds(row0, ROWS_PER_TILE)])
  pltpu.sync_copy(srcf.at[pl.ds(base, GRP)], sidx[0])
  for b in range(NBUF):
    pltpu.sync_copy(dstf.at[pl.ds(base + b * CH, CH)], didx[0][b])
  plsc.subcore_barrier()

  def pair(k2, carry):
    for s in (0, 1):
      g = 2 * k2 + s
      off_n = base + (g + 1) * GRP
      ipf = [pltpu.async_copy(srcf.at[pl.ds(off_n, GRP)], sidx[1 - s], isem)]
      for b in range(NBUF):
        ipf.append(pltpu.async_copy(dstf.at[pl.ds(off_n + b * CH, CH)],
                                    didx[1 - s][b], isem))
      off = base + g * GRP
      del off
      gds = [pltpu.async_copy(tab.at[sidx[s].at[pl.ds(b * CH, CH)]],
                              rows[b], gsem) for b in range(NBUF)]
      sds = []
      for b in range(NBUF):
        gds[b].wait()
        sds.append(pltpu.async_copy(rows[b], acc.at[didx[s][b]], ssem,
                                    add=True))
      for d in sds:
        d.wait()
      for d in ipf:
        d.wait()
    return carry

  lax.fori_loop(0, ngroups // 2, pair, 0)
  plsc.subcore_barrier()
  pltpu.sync_copy(acc.at[pl.ds(row0, ROWS_PER_TILE)],
                  out.at[pl.ds(row0, ROWS_PER_TILE)])


def _make_agg():
  @functools.partial(
      pl.kernel,
      out_type=[
          jax.ShapeDtypeStruct((NPAD, D), jnp.float32),
          jax.ShapeDtypeStruct((NPAD, D), jnp.float32),
      ],
      mesh=_MESH,
      scratch_types=[
          pltpu.VMEM_SHARED((NPAD, D), jnp.float32),
          [pltpu.VMEM((GRP,), jnp.int32) for _ in range(2)],
          [[pltpu.VMEM((CH,), jnp.int32) for _ in range(NBUF)]
           for _ in range(2)],
          [pltpu.VMEM((CH, D), jnp.float32) for _ in range(NBUF)],
          pltpu.SemaphoreType.DMA,
          pltpu.SemaphoreType.DMA,
          pltpu.SemaphoreType.DMA,
      ],
  )
  def k(tab_m, tab_t, src0, dst0, src1, dst1, zrow,
        sum_t, sum_m, acc, sidx, didx, rows, gsem, ssem, isem):
    cid = lax.axis_index("c")
    sid = lax.axis_index("s")

    @pl.when(cid == 0)
    def _():
      _agg_direction(sid, tab_m, src0, dst0, sum_t, acc, sidx, didx, rows,
                     gsem, ssem, isem, zrow)

    @pl.when(cid == 1)
    def _():
      _agg_direction(sid, tab_t, src1, dst1, sum_m, acc, sidx, didx, rows,
                     gsem, ssem, isem, zrow)

  return k


def _make_deg():
  nb = 4

  @functools.partial(
      pl.kernel,
      out_type=[
          jax.ShapeDtypeStruct((NPAD, D), jnp.float32),
          jax.ShapeDtypeStruct((NPAD, D), jnp.float32),
      ],
      mesh=_MESH,
      scratch_types=[
          pltpu.VMEM_SHARED((NPAD, D), jnp.float32),
          [[pltpu.VMEM((CH,), jnp.int32) for _ in range(nb)]
           for _ in range(2)],
          pltpu.VMEM((CH, D), jnp.float32),
          pltpu.SemaphoreType.DMA,
          pltpu.SemaphoreType.DMA,
      ],
  )
  def k(dst0, dst1, zrow, ones_hbm, deg_t, deg_m, dacc, didx, ones_v, ssem,
        isem):
    cid = lax.axis_index("c")
    sid = lax.axis_index("s")
    row0 = sid * ROWS_PER_TILE
    base = sid * E_TILE
    ngroups = N_CHUNKS // nb

    def one(dstf, dout):
      pltpu.sync_copy(zrow, dacc.at[pl.ds(row0, ROWS_PER_TILE)])
      pltpu.sync_copy(ones_hbm, ones_v)
      for b in range(nb):
        pltpu.sync_copy(dstf.at[pl.ds(base + b * CH, CH)], didx[0][b])
      plsc.subcore_barrier()

      def pair(k2, carry):
        for s in (0, 1):
          g = 2 * k2 + s
          off_n = base + (g + 1) * nb * CH
          ipf = [pltpu.async_copy(dstf.at[pl.ds(off_n + b * CH, CH)],
                                  didx[1 - s][b], isem) for b in range(nb)]
          sds = [pltpu.async_copy(ones_v, dacc.at[didx[s][b]], ssem,
                                  add=True) for b in range(nb)]
          for d in sds:
            d.wait()
          for d in ipf:
            d.wait()
        return carry

      lax.fori_loop(0, ngroups // 2, pair, 0)
      plsc.subcore_barrier()
      pltpu.sync_copy(dacc.at[pl.ds(row0, ROWS_PER_TILE)],
                      dout.at[pl.ds(row0, ROWS_PER_TILE)])

    @pl.when(cid == 0)
    def _():
      one(dst0, deg_t)

    @pl.when(cid == 1)
    def _():
      one(dst1, deg_m)

  return k


def _make_label_gather():
  @functools.partial(
      pl.kernel,
      out_type=[
          jax.ShapeDtypeStruct((LBL_PAD, D), jnp.float32),
          jax.ShapeDtypeStruct((LBL_PAD, D), jnp.float32),
      ],
      mesh=_MESH,
      scratch_types=[
          pltpu.VMEM((LBL_TILE,), jnp.int32),
          pltpu.VMEM((LBL_TILE,), jnp.int32),
          pltpu.VMEM((CH, D), jnp.float32),
          pltpu.VMEM((CH, D), jnp.float32),
          pltpu.SemaphoreType.DMA,
          pltpu.SemaphoreType.DMA,
      ],
  )
  def k(tab_t, tab_m, idx0, idx1, ef_t, ef_m, iv0, iv1, rt, rm, sem_t, sem_m):
    cid = lax.axis_index("c")
    sid = lax.axis_index("s")
    wid = cid * NS + sid
    base = wid * LBL_TILE
    pltpu.sync_copy(idx0.at[pl.ds(base, LBL_TILE)], iv0)
    pltpu.sync_copy(idx1.at[pl.ds(base, LBL_TILE)], iv1)

    def step(j, carry):
      off = base + j * CH
      dt = pltpu.async_copy(tab_t.at[iv0.at[pl.ds(j * CH, CH)]], rt, sem_t)
      dm = pltpu.async_copy(tab_m.at[iv1.at[pl.ds(j * CH, CH)]], rm, sem_m)
      dt.wait()
      pltpu.sync_copy(rt, ef_t.at[pl.ds(off, CH)])
      dm.wait()
      pltpu.sync_copy(rm, ef_m.at[pl.ds(off, CH)])
      return carry

    lax.fori_loop(0, LBL_CHUNKS, step, 0)

  return k


_BLK = 400
_GRID = N // _BLK


def _proj_body(x_ref, w_ref, b_ref, emb_ref, o_ref):
  o_ref[...] = (jnp.dot(x_ref[...], w_ref[...],
                        preferred_element_type=jnp.float32)
                + b_ref[...] + emb_ref[...])


def _proj(thesis_x, w, b, emb):
  return pl.pallas_call(
      _proj_body,
      grid=(_GRID,),
      in_specs=[
          pl.BlockSpec((_BLK, DIN), lambda i: (i, 0)),
          pl.BlockSpec((DIN, D), lambda i: (0, 0)),
          pl.BlockSpec((1, D), lambda i: (0, 0)),
          pl.BlockSpec((_BLK, D), lambda i: (i, 0)),
      ],
      out_specs=pl.BlockSpec((_BLK, D), lambda i: (i, 0)),
      out_shape=jax.ShapeDtypeStruct((N, D), jnp.float32),
  )(thesis_x, w, b, emb)


def _layer_body(relu, st_ref, dt_ref, ht_ref, wlt_ref, blt_ref, wrt_ref,
                sm_ref, dm_ref, hm_ref, wlm_ref, blm_ref, wrm_ref,
                t_ref, m_ref):
  def one(s_ref, d_ref, h_ref, wl_ref, bl_ref, wr_ref, o_ref):
    mean = s_ref[...] / jnp.maximum(d_ref[...][:, :1], 1.0)
    r = (jnp.dot(mean, wl_ref[...], preferred_element_type=jnp.float32)
         + bl_ref[...]
         + jnp.dot(h_ref[...], wr_ref[...],
                   preferred_element_type=jnp.float32))
    o_ref[...] = jnp.maximum(r, 0.0) if relu else r

  one(st_ref, dt_ref, ht_ref, wlt_ref, blt_ref, wrt_ref, t_ref)
  one(sm_ref, dm_ref, hm_ref, wlm_ref, blm_ref, wrm_ref, m_ref)


def _layer(relu, sum_t, deg_t, h_t, wl_t, bl_t, wr_t,
           sum_m, deg_m, h_m, wl_m, bl_m, wr_m):
  blk = pl.BlockSpec((_BLK, D), lambda i: (i, 0))
  wfull = pl.BlockSpec((D, D), lambda i: (0, 0))
  bfull = pl.BlockSpec((1, D), lambda i: (0, 0))
  return pl.pallas_call(
      functools.partial(_layer_body, relu),
      grid=(_GRID,),
      in_specs=[blk, blk, blk, wfull, bfull, wfull,
                blk, blk, blk, wfull, bfull, wfull],
      out_specs=[blk, blk],
      out_shape=[jax.ShapeDtypeStruct((N, D), jnp.float32),
                 jax.ShapeDtypeStruct((N, D), jnp.float32)],
  )(sum_t, deg_t, h_t, wl_t, bl_t, wr_t, sum_m, deg_m, h_m, wl_m, bl_m, wr_m)


def _dot_body(a_ref, b_ref, o_ref):
  o_ref[...] = jnp.sum(a_ref[...] * b_ref[...], axis=1, keepdims=True)


def _edge_dot(ef_t, ef_m):
  blk = 512
  return pl.pallas_call(
      _dot_body,
      grid=(LBL_PAD // blk,),
      in_specs=[pl.BlockSpec((blk, D), lambda i: (i, 0)),
                pl.BlockSpec((blk, D), lambda i: (i, 0))],
      out_specs=pl.BlockSpec((blk, 1), lambda i: (i, 0)),
      out_shape=jax.ShapeDtypeStruct((LBL_PAD, 1), jnp.float32),
  )(ef_t, ef_m)


def kernel(thesis_x, thesis_node_id, mentor_node_id, edge_index_t2m,
           edge_index_m2t, edge_label_index, W_lin, b_lin, emb_thesis,
           emb_mentor, Wl_t2m_0, bl_t2m_0, Wr_t2m_0, Wl_m2t_0, bl_m2t_0,
           Wr_m2t_0, Wl_t2m_1, bl_t2m_1, Wr_t2m_1, Wl_m2t_1, bl_m2t_1,
           Wr_m2t_1):
  pad_e = E_PAD + 512 - E

  def padf(a, fill):
    return jnp.concatenate([a, jnp.full((pad_e,), fill, jnp.int32)])

  src0 = padf(edge_index_m2t[0], 0)
  dst0 = padf(edge_index_m2t[1], N)
  src1 = padf(edge_index_t2m[0], 0)
  dst1 = padf(edge_index_t2m[1], N)
  pad_l = LBL_PAD - ELBL
  eli0 = jnp.concatenate(
      [edge_label_index[0], jnp.zeros((pad_l,), jnp.int32)])
  eli1 = jnp.concatenate(
      [edge_label_index[1], jnp.zeros((pad_l,), jnp.int32)])
  zrow = jnp.zeros((ROWS_PER_TILE, D), jnp.float32)
  ones_hbm = jnp.ones((CH, D), jnp.float32)
  b2 = b_lin.reshape(1, D)

  h_t = _proj(thesis_x, W_lin, b2, emb_thesis)
  h_m = emb_mentor

  deg_t, deg_m = _make_deg()(dst0, dst1, zrow, ones_hbm)

  agg = _make_agg()
  sum_t0, sum_m0 = agg(h_m, h_t, src0, dst0, src1, dst1, zrow)
  t0, m0 = _layer(True, sum_t0, deg_t, h_t, Wl_m2t_0,
                  bl_m2t_0.reshape(1, D), Wr_m2t_0,
                  sum_m0, deg_m, h_m, Wl_t2m_0,
                  bl_t2m_0.reshape(1, D), Wr_t2m_0)

  sum_t1, sum_m1 = agg(m0, t0, src0, dst0, src1, dst1, zrow)
  t1, m1 = _layer(False, sum_t1, deg_t, t0, Wl_m2t_1,
                  bl_m2t_1.reshape(1, D), Wr_m2t_1,
                  sum_m1, deg_m, m0, Wl_t2m_1,
                  bl_t2m_1.reshape(1, D), Wr_t2m_1)

  ef_t, ef_m = _make_label_gather()(t1, m1, eli0, eli1)
  scores = _edge_dot(ef_t, ef_m)
  return scores[:ELBL, 0]


# speedup vs baseline: 1.0939x; 1.0939x over previous
import functools

import jax
import jax.numpy as jnp
from jax import lax
from jax.experimental import pallas as pl
from jax.experimental.pallas import tpu as pltpu
from jax.experimental.pallas import tpu_sc as plsc

N = 10000
E = 320000
ELBL = 20000
DIN = 384
D = 128

NC, NS = 2, 16
CH = 128
NPAD = 10240
ROWS_PER_TILE = NPAD // NS
E_TILE = 20096
E_PAD = E_TILE * NS
N_CHUNKS = E_TILE // CH
LBL_TILE = 640
LBL_PAD = LBL_TILE * NC * NS
LBL_CHUNKS = LBL_TILE // CH

_MESH = plsc.VectorSubcoreMesh(
    core_axis_name="c", subcore_axis_name="s", num_cores=NC, num_subcores=NS)


def _agg_direction(sid, tab, src, dst, out, acc, idx_s, idx_d, rows, gsem,
                   ssem, zrow, deg_out, dacc, ones_v, zdeg, ones_hbm):
  row0 = sid * ROWS_PER_TILE
  pltpu.sync_copy(zrow, acc.at[pl.ds(row0, ROWS_PER_TILE)])
  if deg_out is not None:
    pltpu.sync_copy(zdeg, dacc.at[pl.ds(row0, ROWS_PER_TILE)])
    pltpu.sync_copy(ones_hbm, ones_v)
  plsc.subcore_barrier()

  base = sid * E_TILE

  def step(j, carry):
    off = base + j * CH
    pltpu.sync_copy(src.at[pl.ds(off, CH)], idx_s)
    pltpu.sync_copy(dst.at[pl.ds(off, CH)], idx_d)
    pltpu.async_copy(tab.at[idx_s], rows, gsem).wait()
    pltpu.async_copy(rows, acc.at[idx_d], ssem, add=True).wait()
    if deg_out is not None:
      pltpu.async_copy(ones_v, dacc.at[idx_d], ssem, add=True).wait()
    return carry

  lax.fori_loop(0, N_CHUNKS, step, 0)
  plsc.subcore_barrier()
  pltpu.sync_copy(acc.at[pl.ds(row0, ROWS_PER_TILE)],
                  out.at[pl.ds(row0, ROWS_PER_TILE)])
  if deg_out is not None:
    pltpu.sync_copy(dacc.at[pl.ds(row0, ROWS_PER_TILE)],
                    deg_out.at[pl.ds(row0, ROWS_PER_TILE)])


def _make_deg():
  @functools.partial(
      pl.kernel,
      out_type=[
          jax.ShapeDtypeStruct((NPAD, D), jnp.float32),
          jax.ShapeDtypeStruct((NPAD, D), jnp.float32),
      ],
      mesh=_MESH,
      scratch_types=[
          pltpu.VMEM_SHARED((NPAD, D), jnp.float32),
          pltpu.VMEM((CH,), jnp.int32),
          pltpu.VMEM((CH, D), jnp.float32),
          pltpu.SemaphoreType.DMA,
      ],
  )
  def k(dst0, dst1, zdeg, ones_hbm, deg_t, deg_m, dacc, idx_d, ones_v, ssem):
    cid = lax.axis_index("c")
    sid = lax.axis_index("s")
    row0 = sid * ROWS_PER_TILE
    base = sid * E_TILE

    def one(dst, dout):
      pltpu.sync_copy(zdeg, dacc.at[pl.ds(row0, ROWS_PER_TILE)])
      pltpu.sync_copy(ones_hbm, ones_v)
      plsc.subcore_barrier()

      def step(j, carry):
        off = base + j * CH
        pltpu.sync_copy(dst.at[pl.ds(off, CH)], idx_d)
        pltpu.async_copy(ones_v, dacc.at[idx_d], ssem, add=True).wait()
        return carry

      lax.fori_loop(0, N_CHUNKS, step, 0)
      plsc.subcore_barrier()
      pltpu.sync_copy(dacc.at[pl.ds(row0, ROWS_PER_TILE)],
                      dout.at[pl.ds(row0, ROWS_PER_TILE)])

    @pl.when(cid == 0)
    def _():
      one(dst0, deg_t)

    @pl.when(cid == 1)
    def _():
      one(dst1, deg_m)

  return k


def _make_agg_l1():
  @functools.partial(
      pl.kernel,
      out_type=[
          jax.ShapeDtypeStruct((NPAD, D), jnp.float32),
          jax.ShapeDtypeStruct((NPAD, D), jnp.float32),
      ],
      mesh=_MESH,
      scratch_types=[
          pltpu.VMEM_SHARED((NPAD, D), jnp.float32),
          pltpu.VMEM((CH,), jnp.int32),
          pltpu.VMEM((CH,), jnp.int32),
          pltpu.VMEM((CH, D), jnp.float32),
          pltpu.SemaphoreType.DMA,
          pltpu.SemaphoreType.DMA,
      ],
  )
  def k(tab_m, tab_t, src0, dst0, src1, dst1, zrow,
        sum_t, sum_m, acc, idx_s, idx_d, rows, gsem, ssem):
    cid = lax.axis_index("c")
    sid = lax.axis_index("s")

    @pl.when(cid == 0)
    def _():
      _agg_direction(sid, tab_m, src0, dst0, sum_t, acc, idx_s, idx_d, rows,
                     gsem, ssem, zrow, None, None, None, None, None)

    @pl.when(cid == 1)
    def _():
      _agg_direction(sid, tab_t, src1, dst1, sum_m, acc, idx_s, idx_d, rows,
                     gsem, ssem, zrow, None, None, None, None, None)

  return k


def _make_label_gather():
  @functools.partial(
      pl.kernel,
      out_type=[
          jax.ShapeDtypeStruct((LBL_PAD, D), jnp.float32),
          jax.ShapeDtypeStruct((LBL_PAD, D), jnp.float32),
      ],
      mesh=_MESH,
      scratch_types=[
          pltpu.VMEM((CH,), jnp.int32),
          pltpu.VMEM((CH, D), jnp.float32),
          pltpu.SemaphoreType.DMA,
      ],
  )
  def k(tab_t, tab_m, idx0, idx1, ef_t, ef_m, idx_v, rows, gsem):
    cid = lax.axis_index("c")
    sid = lax.axis_index("s")
    wid = cid * NS + sid
    base = wid * LBL_TILE

    def step(j, carry):
      off = base + j * CH
      pltpu.sync_copy(idx0.at[pl.ds(off, CH)], idx_v)
      pltpu.async_copy(tab_t.at[idx_v], rows, gsem).wait()
      pltpu.sync_copy(rows, ef_t.at[pl.ds(off, CH)])
      pltpu.sync_copy(idx1.at[pl.ds(off, CH)], idx_v)
      pltpu.async_copy(tab_m.at[idx_v], rows, gsem).wait()
      pltpu.sync_copy(rows, ef_m.at[pl.ds(off, CH)])
      return carry

    lax.fori_loop(0, LBL_CHUNKS, step, 0)

  return k


_BLK = 400
_GRID = N // _BLK


def _proj_body(x_ref, w_ref, b_ref, emb_ref, o_ref):
  o_ref[...] = (jnp.dot(x_ref[...], w_ref[...],
                        preferred_element_type=jnp.float32)
                + b_ref[...] + emb_ref[...])


def _proj(thesis_x, w, b, emb):
  return pl.pallas_call(
      _proj_body,
      grid=(_GRID,),
      in_specs=[
          pl.BlockSpec((_BLK, DIN), lambda i: (i, 0)),
          pl.BlockSpec((DIN, D), lambda i: (0, 0)),
          pl.BlockSpec((1, D), lambda i: (0, 0)),
          pl.BlockSpec((_BLK, D), lambda i: (i, 0)),
      ],
      out_specs=pl.BlockSpec((_BLK, D), lambda i: (i, 0)),
      out_shape=jax.ShapeDtypeStruct((N, D), jnp.float32),
  )(thesis_x, w, b, emb)


def _layer_body(relu, st_ref, dt_ref, ht_ref, wlt_ref, blt_ref, wrt_ref,
                sm_ref, dm_ref, hm_ref, wlm_ref, blm_ref, wrm_ref,
                t_ref, m_ref):
  def one(s_ref, d_ref, h_ref, wl_ref, bl_ref, wr_ref, o_ref):
    mean = s_ref[...] / jnp.maximum(d_ref[...][:, :1], 1.0)
    r = (jnp.dot(mean, wl_ref[...], preferred_element_type=jnp.float32)
         + bl_ref[...]
         + jnp.dot(h_ref[...], wr_ref[...],
                   preferred_element_type=jnp.float32))
    o_ref[...] = jnp.maximum(r, 0.0) if relu else r

  one(st_ref, dt_ref, ht_ref, wlt_ref, blt_ref, wrt_ref, t_ref)
  one(sm_ref, dm_ref, hm_ref, wlm_ref, blm_ref, wrm_ref, m_ref)


def _layer(relu, sum_t, deg_t, h_t, wl_t, bl_t, wr_t,
           sum_m, deg_m, h_m, wl_m, bl_m, wr_m):
  blk = pl.BlockSpec((_BLK, D), lambda i: (i, 0))
  deg = pl.BlockSpec((_BLK, D), lambda i: (i, 0))
  wfull = pl.BlockSpec((D, D), lambda i: (0, 0))
  bfull = pl.BlockSpec((1, D), lambda i: (0, 0))
  return pl.pallas_call(
      functools.partial(_layer_body, relu),
      grid=(_GRID,),
      in_specs=[blk, deg, blk, wfull, bfull, wfull,
                blk, deg, blk, wfull, bfull, wfull],
      out_specs=[blk, blk],
      out_shape=[jax.ShapeDtypeStruct((N, D), jnp.float32),
                 jax.ShapeDtypeStruct((N, D), jnp.float32)],
  )(sum_t, deg_t, h_t, wl_t, bl_t, wr_t, sum_m, deg_m, h_m, wl_m, bl_m, wr_m)


def _dot_body(a_ref, b_ref, o_ref):
  o_ref[...] = jnp.sum(a_ref[...] * b_ref[...], axis=1, keepdims=True)


def _edge_dot(ef_t, ef_m):
  blk = 512
  return pl.pallas_call(
      _dot_body,
      grid=(LBL_PAD // blk,),
      in_specs=[pl.BlockSpec((blk, D), lambda i: (i, 0)),
                pl.BlockSpec((blk, D), lambda i: (i, 0))],
      out_specs=pl.BlockSpec((blk, 1), lambda i: (i, 0)),
      out_shape=jax.ShapeDtypeStruct((LBL_PAD, 1), jnp.float32),
  )(ef_t, ef_m)


def kernel(thesis_x, thesis_node_id, mentor_node_id, edge_index_t2m,
           edge_index_m2t, edge_label_index, W_lin, b_lin, emb_thesis,
           emb_mentor, Wl_t2m_0, bl_t2m_0, Wr_t2m_0, Wl_m2t_0, bl_m2t_0,
           Wr_m2t_0, Wl_t2m_1, bl_t2m_1, Wr_t2m_1, Wl_m2t_1, bl_m2t_1,
           Wr_m2t_1):
  pad_e = E_PAD - E
  src0 = jnp.concatenate([edge_index_m2t[0],
                          jnp.zeros((pad_e,), jnp.int32)])
  dst0 = jnp.concatenate([edge_index_m2t[1],
                          jnp.full((pad_e,), N, jnp.int32)])
  src1 = jnp.concatenate([edge_index_t2m[0],
                          jnp.zeros((pad_e,), jnp.int32)])
  dst1 = jnp.concatenate([edge_index_t2m[1],
                          jnp.full((pad_e,), N, jnp.int32)])
  pad_l = LBL_PAD - ELBL
  eli0 = jnp.concatenate([edge_label_index[0],
                          jnp.zeros((pad_l,), jnp.int32)])
  eli1 = jnp.concatenate([edge_label_index[1],
                          jnp.zeros((pad_l,), jnp.int32)])
  zrow = jnp.zeros((ROWS_PER_TILE, D), jnp.float32)
  zdeg = jnp.zeros((ROWS_PER_TILE, D), jnp.float32)
  ones_hbm = jnp.ones((CH, D), jnp.float32)
  b2 = b_lin.reshape(1, D)

  h_t = _proj(thesis_x, W_lin, b2, emb_thesis)
  h_m = emb_mentor

  agg = _make_agg_l1()
  sum_t0, sum_m0 = agg(h_m, h_t, src0, dst0, src1, dst1, zrow)
  deg_t, deg_m = _make_deg()(dst0, dst1, zdeg, ones_hbm)
  t0, m0 = _layer(True, sum_t0, deg_t, h_t, Wl_m2t_0,
                  bl_m2t_0.reshape(1, D), Wr_m2t_0,
                  sum_m0, deg_m, h_m, Wl_t2m_0,
                  bl_t2m_0.reshape(1, D), Wr_t2m_0)

  sum_t1, sum_m1 = agg(m0, t0, src0, dst0, src1, dst1, zrow)
  t1, m1 = _layer(False, sum_t1, deg_t, t0, Wl_m2t_1,
                  bl_m2t_1.reshape(1, D), Wr_m2t_1,
                  sum_m1, deg_m, m0, Wl_t2m_1,
                  bl_t2m_1.reshape(1, D), Wr_t2m_1)

  ef_t, ef_m = _make_label_gather()(t1, m1, eli0, eli1)
  scores = _edge_dot(ef_t, ef_m)
  return scores[:ELBL, 0]
